# baseline (speedup 1.0000x reference)
import jax
import jax.numpy as jnp
from jax.experimental import pallas as pl
from jax.experimental.pallas import tpu as pltpu

E = 64
T = 2048
D = 768
F = 2048
TT = 256
NT = T // TT + E


def _route_kernel(x_ref, gate_ref, p_ref, te_ref, act_ref):
    x = x_ref[...]
    gate = gate_ref[...]
    logits = jnp.dot(x, gate, preferred_element_type=jnp.float32)
    m = jnp.max(logits, axis=1, keepdims=True)
    e_iota = jax.lax.broadcasted_iota(jnp.int32, (T, E), 1)
    eid = jnp.min(jnp.where(logits == m, e_iota, E), axis=1, keepdims=True)
    onehot = (e_iota == eid).astype(jnp.bfloat16)

    r_iota = jax.lax.broadcasted_iota(jnp.int32, (T, T), 0)
    c_iota = jax.lax.broadcasted_iota(jnp.int32, (T, T), 1)
    ltri = (c_iota < r_iota).astype(jnp.bfloat16)
    before = jnp.dot(ltri, onehot, preferred_element_type=jnp.float32)
    rank = jnp.sum(before * onehot.astype(jnp.float32), axis=1, keepdims=True)

    counts = jnp.sum(onehot.astype(jnp.float32), axis=0, keepdims=True)
    ntiles = jnp.floor((counts + (TT - 1)) * (1.0 / TT))
    tri_inc = (jax.lax.broadcasted_iota(jnp.int32, (E, E), 0)
               <= jax.lax.broadcasted_iota(jnp.int32, (E, E), 1)).astype(jnp.bfloat16)
    cum_inc = jnp.dot(ntiles.astype(jnp.bfloat16), tri_inc,
                      preferred_element_type=jnp.float32)
    cum_exc = cum_inc - ntiles

    base_t = jnp.sum(onehot.astype(jnp.float32) * cum_exc, axis=1, keepdims=True)
    p_ref[...] = (base_t * TT + rank).astype(jnp.int32)

    i_iota = jax.lax.broadcasted_iota(jnp.int32, (NT, E), 0).astype(jnp.float32)
    te_raw = jnp.sum((i_iota >= cum_inc).astype(jnp.int32), axis=1, keepdims=True)
    e64 = jax.lax.broadcasted_iota(jnp.int32, (1, E), 1)
    last_e = jnp.max(jnp.where(counts > 0, e64, 0), axis=1, keepdims=True)
    te_ref[...] = jnp.minimum(te_raw, last_e)
    act_ref[...] = (te_raw < E).astype(jnp.int32)


def _ffn_kernel(te_ref, act_ref, p_ref, x_ref, w0_ref, w1_ref, wo_ref,
                out_ref):
    i = pl.program_id(0)

    @pl.when(i == 0)
    def _init():
        out_ref[...] = jnp.zeros_like(out_ref)

    @pl.when(act_ref[i] == 1)
    def _compute():
        p = p_ref[...]
        slot = jax.lax.broadcasted_iota(jnp.int32, (T, TT), 1) + i * TT
        gt = (p == slot).astype(jnp.bfloat16)
        xt = jax.lax.dot_general(gt, x_ref[...], (((0,), (0,)), ((), ())),
                                 preferred_element_type=jnp.float32)
        xtb = xt.astype(jnp.bfloat16)
        h0 = jnp.dot(xtb, w0_ref[0].astype(jnp.bfloat16),
                     preferred_element_type=jnp.float32)
        h1 = jnp.dot(xtb, w1_ref[0].astype(jnp.bfloat16),
                     preferred_element_type=jnp.float32)
        h = (h0 * jax.nn.sigmoid(h0) * h1).astype(jnp.bfloat16)
        o = jnp.dot(h, wo_ref[0].astype(jnp.bfloat16),
                    preferred_element_type=jnp.float32)
        out_ref[...] += jnp.dot(gt, o.astype(jnp.bfloat16),
                                preferred_element_type=jnp.float32)


def kernel(x, gate_kernel, w0_kernel, w1_kernel, wo_kernel):
    xs = x.shape
    x2d = jnp.reshape(x, (T, D))

    p, te, act = pl.pallas_call(
        _route_kernel,
        out_shape=[
            jax.ShapeDtypeStruct((T, 1), jnp.int32),
            jax.ShapeDtypeStruct((NT, 1), jnp.int32),
            jax.ShapeDtypeStruct((NT, 1), jnp.int32),
        ],
    )(x2d, gate_kernel)
    te1d = te.reshape(NT)
    act1d = act.reshape(NT)

    xb = x2d.astype(jnp.bfloat16)

    grid_spec = pltpu.PrefetchScalarGridSpec(
        num_scalar_prefetch=2,
        grid=(NT,),
        in_specs=[
            pl.BlockSpec((T, 1), lambda i, te, act: (0, 0)),
            pl.BlockSpec((T, D), lambda i, te, act: (0, 0)),
            pl.BlockSpec((1, D, F), lambda i, te, act: (te[i], 0, 0)),
            pl.BlockSpec((1, D, F), lambda i, te, act: (te[i], 0, 0)),
            pl.BlockSpec((1, F, D), lambda i, te, act: (te[i], 0, 0)),
        ],
        out_specs=pl.BlockSpec((T, D), lambda i, te, act: (0, 0)),
    )
    out = pl.pallas_call(
        _ffn_kernel,
        grid_spec=grid_spec,
        out_shape=jax.ShapeDtypeStruct((T, D), jnp.float32),
        compiler_params=pltpu.CompilerParams(
            vmem_limit_bytes=100 * 1024 * 1024),
    )(te1d, act1d, p, xb, w0_kernel, w1_kernel, wo_kernel)

    return jnp.reshape(out, xs)


# speedup vs baseline: 1.0325x; 1.0325x over previous
import jax
import jax.numpy as jnp
from jax.experimental import pallas as pl
from jax.experimental.pallas import tpu as pltpu

E = 64
T = 2048
D = 768
F = 2048
TT = 256
NT = T // TT + E


def _route_kernel(x_ref, gate_ref, p_ref, te_ref, act_ref):
    x = x_ref[...]
    gate = gate_ref[...]
    logits = jnp.dot(x, gate, preferred_element_type=jnp.float32)
    m = jnp.max(logits, axis=1, keepdims=True)
    e_iota = jax.lax.broadcasted_iota(jnp.int32, (T, E), 1)
    eid = jnp.min(jnp.where(logits == m, e_iota, E), axis=1, keepdims=True)
    onehot = (e_iota == eid).astype(jnp.bfloat16)

    r_iota = jax.lax.broadcasted_iota(jnp.int32, (T, T), 0)
    c_iota = jax.lax.broadcasted_iota(jnp.int32, (T, T), 1)
    ltri = (c_iota < r_iota).astype(jnp.bfloat16)
    before = jnp.dot(ltri, onehot, preferred_element_type=jnp.float32)
    rank = jnp.sum(before * onehot.astype(jnp.float32), axis=1, keepdims=True)

    counts = jnp.sum(onehot.astype(jnp.float32), axis=0, keepdims=True)
    ntiles = jnp.floor((counts + (TT - 1)) * (1.0 / TT))
    tri_inc = (jax.lax.broadcasted_iota(jnp.int32, (E, E), 0)
               <= jax.lax.broadcasted_iota(jnp.int32, (E, E), 1)).astype(jnp.bfloat16)
    cum_inc = jnp.dot(ntiles.astype(jnp.bfloat16), tri_inc,
                      preferred_element_type=jnp.float32)
    cum_exc = cum_inc - ntiles

    base_t = jnp.sum(onehot.astype(jnp.float32) * cum_exc, axis=1, keepdims=True)
    p_ref[...] = (base_t * TT + rank).astype(jnp.int32)

    i_iota = jax.lax.broadcasted_iota(jnp.int32, (NT, E), 0).astype(jnp.float32)
    te_raw = jnp.sum((i_iota >= cum_inc).astype(jnp.int32), axis=1, keepdims=True)
    e64 = jax.lax.broadcasted_iota(jnp.int32, (1, E), 1)
    last_e = jnp.max(jnp.where(counts > 0, e64, 0), axis=1, keepdims=True)
    te_ref[...] = jnp.minimum(te_raw, last_e)
    act_ref[...] = (te_raw < E).astype(jnp.int32)


def _ffn_kernel(te_ref, act_ref, p_ref, x_ref, w0_ref, w1_ref, wo_ref,
                out_ref):
    i = pl.program_id(0)

    @pl.when(i == 0)
    def _init():
        out_ref[...] = jnp.zeros_like(out_ref)

    @pl.when(act_ref[i] == 1)
    def _compute():
        p = p_ref[...]
        slot = jax.lax.broadcasted_iota(jnp.int32, (T, TT), 1) + i * TT
        gt = (p == slot).astype(jnp.float32)
        xt = jax.lax.dot_general(gt, x_ref[...], (((0,), (0,)), ((), ())),
                                 preferred_element_type=jnp.float32)
        gtb = gt.astype(jnp.bfloat16)
        xtb = xt.astype(jnp.bfloat16)
        h0 = jnp.dot(xtb, w0_ref[0].astype(jnp.bfloat16),
                     preferred_element_type=jnp.float32)
        h1 = jnp.dot(xtb, w1_ref[0].astype(jnp.bfloat16),
                     preferred_element_type=jnp.float32)
        h = (h0 * jax.nn.sigmoid(h0) * h1).astype(jnp.bfloat16)
        o = jnp.dot(h, wo_ref[0].astype(jnp.bfloat16),
                    preferred_element_type=jnp.float32)
        out_ref[...] += jnp.dot(gtb, o.astype(jnp.bfloat16),
                                preferred_element_type=jnp.float32)


def kernel(x, gate_kernel, w0_kernel, w1_kernel, wo_kernel):
    xs = x.shape
    x2d = jnp.reshape(x, (T, D))

    p, te, act = pl.pallas_call(
        _route_kernel,
        out_shape=[
            jax.ShapeDtypeStruct((T, 1), jnp.int32),
            jax.ShapeDtypeStruct((NT, 1), jnp.int32),
            jax.ShapeDtypeStruct((NT, 1), jnp.int32),
        ],
    )(x2d, gate_kernel)
    te1d = te.reshape(NT)
    act1d = act.reshape(NT)

    grid_spec = pltpu.PrefetchScalarGridSpec(
        num_scalar_prefetch=2,
        grid=(NT,),
        in_specs=[
            pl.BlockSpec((T, 1), lambda i, te, act: (0, 0)),
            pl.BlockSpec((T, D), lambda i, te, act: (0, 0)),
            pl.BlockSpec((1, D, F), lambda i, te, act: (te[i], 0, 0)),
            pl.BlockSpec((1, D, F), lambda i, te, act: (te[i], 0, 0)),
            pl.BlockSpec((1, F, D), lambda i, te, act: (te[i], 0, 0)),
        ],
        out_specs=pl.BlockSpec((T, D), lambda i, te, act: (0, 0)),
    )
    out = pl.pallas_call(
        _ffn_kernel,
        grid_spec=grid_spec,
        out_shape=jax.ShapeDtypeStruct((T, D), jnp.float32),
        compiler_params=pltpu.CompilerParams(
            vmem_limit_bytes=100 * 1024 * 1024),
    )(te1d, act1d, p, x2d, w0_kernel, w1_kernel, wo_kernel)

    return jnp.reshape(out, xs)


# speedup vs baseline: 1.1008x; 1.0662x over previous
import jax
import jax.numpy as jnp
from jax.experimental import pallas as pl
from jax.experimental.pallas import tpu as pltpu

E = 64
T = 2048
D = 768
F = 2048
TT = 128
NT = T // TT + E


def _route_kernel(x_ref, gate_ref, p_ref, te_ref, act_ref):
    x = x_ref[...]
    gate = gate_ref[...]
    logits = jnp.dot(x, gate, preferred_element_type=jnp.float32)
    m = jnp.max(logits, axis=1, keepdims=True)
    e_iota = jax.lax.broadcasted_iota(jnp.int32, (T, E), 1)
    eid = jnp.min(jnp.where(logits == m, e_iota, E), axis=1, keepdims=True)
    onehot = (e_iota == eid).astype(jnp.bfloat16)

    r_iota = jax.lax.broadcasted_iota(jnp.int32, (T, T), 0)
    c_iota = jax.lax.broadcasted_iota(jnp.int32, (T, T), 1)
    ltri = (c_iota < r_iota).astype(jnp.bfloat16)
    before = jnp.dot(ltri, onehot, preferred_element_type=jnp.float32)
    rank = jnp.sum(before * onehot.astype(jnp.float32), axis=1, keepdims=True)

    counts = jnp.sum(onehot.astype(jnp.float32), axis=0, keepdims=True)
    ntiles = jnp.floor((counts + (TT - 1)) * (1.0 / TT))
    tri_inc = (jax.lax.broadcasted_iota(jnp.int32, (E, E), 0)
               <= jax.lax.broadcasted_iota(jnp.int32, (E, E), 1)).astype(jnp.bfloat16)
    cum_inc = jnp.dot(ntiles.astype(jnp.bfloat16), tri_inc,
                      preferred_element_type=jnp.float32)
    cum_exc = cum_inc - ntiles

    base_t = jnp.sum(onehot.astype(jnp.float32) * cum_exc, axis=1, keepdims=True)
    p_ref[...] = (base_t * TT + rank).astype(jnp.int32)

    i_iota = jax.lax.broadcasted_iota(jnp.int32, (NT, E), 0).astype(jnp.float32)
    te_raw = jnp.sum((i_iota >= cum_inc).astype(jnp.int32), axis=1, keepdims=True)
    e64 = jax.lax.broadcasted_iota(jnp.int32, (1, E), 1)
    last_e = jnp.max(jnp.where(counts > 0, e64, 0), axis=1, keepdims=True)
    te_ref[...] = jnp.minimum(te_raw, last_e)
    act_ref[...] = (te_raw < E).astype(jnp.int32)


def _ffn_kernel(te_ref, act_ref, p_ref, x_ref, w0_ref, w1_ref, wo_ref,
                out_ref):
    i = pl.program_id(0)

    @pl.when(i == 0)
    def _init():
        out_ref[...] = jnp.zeros_like(out_ref)

    @pl.when(act_ref[i] == 1)
    def _compute():
        p = p_ref[...]
        slot = jax.lax.broadcasted_iota(jnp.int32, (T, TT), 1) + i * TT
        gt = (p == slot).astype(jnp.float32)
        xt = jax.lax.dot_general(gt, x_ref[...], (((0,), (0,)), ((), ())),
                                 preferred_element_type=jnp.float32)
        gtb = gt.astype(jnp.bfloat16)
        xtb = xt.astype(jnp.bfloat16)
        h0 = jnp.dot(xtb, w0_ref[0].astype(jnp.bfloat16),
                     preferred_element_type=jnp.float32)
        h1 = jnp.dot(xtb, w1_ref[0].astype(jnp.bfloat16),
                     preferred_element_type=jnp.float32)
        h = (h0 * jax.nn.sigmoid(h0) * h1).astype(jnp.bfloat16)
        o = jnp.dot(h, wo_ref[0].astype(jnp.bfloat16),
                    preferred_element_type=jnp.float32)
        out_ref[...] += jnp.dot(gtb, o.astype(jnp.bfloat16),
                                preferred_element_type=jnp.float32)


def kernel(x, gate_kernel, w0_kernel, w1_kernel, wo_kernel):
    xs = x.shape
    x2d = jnp.reshape(x, (T, D))

    p, te, act = pl.pallas_call(
        _route_kernel,
        out_shape=[
            jax.ShapeDtypeStruct((T, 1), jnp.int32),
            jax.ShapeDtypeStruct((NT, 1), jnp.int32),
            jax.ShapeDtypeStruct((NT, 1), jnp.int32),
        ],
    )(x2d, gate_kernel)
    te1d = te.reshape(NT)
    act1d = act.reshape(NT)

    grid_spec = pltpu.PrefetchScalarGridSpec(
        num_scalar_prefetch=2,
        grid=(NT,),
        in_specs=[
            pl.BlockSpec((T, 1), lambda i, te, act: (0, 0)),
            pl.BlockSpec((T, D), lambda i, te, act: (0, 0)),
            pl.BlockSpec((1, D, F), lambda i, te, act: (te[i], 0, 0)),
            pl.BlockSpec((1, D, F), lambda i, te, act: (te[i], 0, 0)),
            pl.BlockSpec((1, F, D), lambda i, te, act: (te[i], 0, 0)),
        ],
        out_specs=pl.BlockSpec((T, D), lambda i, te, act: (0, 0)),
    )
    out = pl.pallas_call(
        _ffn_kernel,
        grid_spec=grid_spec,
        out_shape=jax.ShapeDtypeStruct((T, D), jnp.float32),
        compiler_params=pltpu.CompilerParams(
            vmem_limit_bytes=100 * 1024 * 1024),
    )(te1d, act1d, p, x2d, w0_kernel, w1_kernel, wo_kernel)

    return jnp.reshape(out, xs)
